# Initial kernel scaffold; baseline (speedup 1.0000x reference)
import jax
import jax.numpy as jnp
from jax.experimental import pallas as pl


def kernel(x, codes, params):
    raise NotImplementedError("write your pallas kernel here")


# speedup vs baseline: 8.5449x; 8.5449x over previous
import functools

import numpy as np
import jax
import jax.numpy as jnp
from jax.experimental import pallas as pl
from jax.experimental.pallas import tpu as pltpu

_GRID_SIZE = 48
_RES = 0.25
_SPACING = 2.0
_H = 128
_K = 16
_NL = 3


def _anchors_np():
    half = (_GRID_SIZE - 1) * _RES / 2.0
    n = int(np.floor(2.0 * half / _SPACING)) + 1
    lin = (np.arange(n) - (n - 1) / 2.0) * _SPACING
    g = np.stack(np.meshgrid(lin, lin, lin, indexing='ij'), axis=-1).reshape(-1, 3)
    return g.astype(np.float32)


def _silu(v):
    return v * (1.0 / (1.0 + jnp.exp(-v)))


def _prep_kernel(codes_ref, wc_ref, bc_ref, wm2_ref, a_ref, t_ref):
    c = codes_ref[0]
    ha = _silu(jnp.dot(c, wc_ref[...], preferred_element_type=jnp.float32)
               + bc_ref[...])
    gs = [jnp.dot(ha, wm2_ref[l], preferred_element_type=jnp.float32)
          for l in range(_NL)]
    G = jnp.concatenate(gs, axis=1)
    ghi = G.astype(jnp.bfloat16)
    glo = (G - ghi.astype(jnp.float32)).astype(jnp.bfloat16)
    apad = a_ref[...].astype(jnp.bfloat16)
    t_ref[0] = jnp.concatenate([ghi, glo, apad], axis=1)


def _main_kernel(x_ref, t_ref, ar_ref, wq_ref, bq_ref, wm1_ref, bmsg_ref,
                 wd2_ref, wx_ref, wu1_ref, wu2_ref, bupd_ref, wout_ref,
                 bout_ref, o_ref, *, tp, na):
    xb = x_ref[0]
    d2 = None
    for c in range(3):
        t = xb[:, c:c + 1] - ar_ref[c:c + 1, :]
        t = t * t
        d2 = t if d2 is None else d2 + t
    iota = jax.lax.broadcasted_iota(jnp.int32, (tp, na), 1)
    ohs = []
    for _ in range(_K):
        mn = jnp.min(d2, axis=1, keepdims=True)
        idxm = jnp.where(d2 <= mn, iota, na)
        amn = jnp.min(idxm, axis=1, keepdims=True)
        oh = iota == amn
        ohs.append(oh.astype(jnp.bfloat16))
        d2 = jnp.where(oh, jnp.float32(3.0e38), d2)
    O = jnp.concatenate(ohs, axis=0)
    GA = jnp.dot(O, t_ref[0], preferred_element_type=jnp.float32)
    Ge = GA[:, 0:3 * _H] + GA[:, 3 * _H:6 * _H]
    NP = GA[:, 6 * _H:6 * _H + 3]

    hq = _silu(jnp.dot(xb, wq_ref[...], preferred_element_type=jnp.float32)
               + bq_ref[...])
    xc = xb
    inv_k = jnp.float32(1.0 / _K)
    for l in range(_NL):
        qp = jnp.dot(hq, wm1_ref[l], preferred_element_type=jnp.float32) \
            + bmsg_ref[l:l + 1, :]
        qpe = jnp.concatenate([qp] * _K, axis=0)
        xce = jnp.concatenate([xc] * _K, axis=0)
        rel = xce - NP
        d2e = jnp.sum(rel * rel, axis=1, keepdims=True)
        m = _silu(qpe + Ge[:, l * _H:(l + 1) * _H] + d2e * wd2_ref[l:l + 1, :])
        agg = jnp.sum(m.reshape(_K, tp, _H), axis=0) * inv_k
        cw = jnp.sum(m * wx_ref[l:l + 1, :], axis=1, keepdims=True)
        xc = xc + jnp.sum((rel * cw).reshape(_K, tp, 3), axis=0) * inv_k
        hq = _silu(jnp.dot(hq, wu1_ref[l], preferred_element_type=jnp.float32)
                   + jnp.dot(agg, wu2_ref[l], preferred_element_type=jnp.float32)
                   + bupd_ref[l:l + 1, :])
    o_ref[0] = jnp.dot(hq, wout_ref[...], preferred_element_type=jnp.float32) \
        + bout_ref[...]


def _run(x, codes, params, interpret=False):
    B, P, _ = x.shape
    A = codes.shape[1]
    H = _H
    anc = _anchors_np()
    apad = np.zeros((A, 128), np.float32)
    apad[:, :3] = anc
    apad_j = jnp.asarray(apad)
    arows_j = jnp.asarray(anc.T.copy())

    wmsg = params['W_msg']
    wm1 = wmsg[:, :H, :]
    wm2 = wmsg[:, H:2 * H, :]
    wd2 = wmsg[:, 2 * H, :]
    wupd = params['W_upd']
    wu1 = wupd[:, :H, :]
    wu2 = wupd[:, H:, :]
    wx = params['W_x'][:, :, 0]
    bq = params['b_q'].reshape(1, H)
    bc = params['b_code'].reshape(1, H)
    bout = params['b_out'].reshape(1, -1)
    nch = bout.shape[1]

    T = pl.pallas_call(
        _prep_kernel,
        grid=(B,),
        in_specs=[
            pl.BlockSpec((1, A, H), lambda b: (b, 0, 0)),
            pl.BlockSpec((H, H), lambda b: (0, 0)),
            pl.BlockSpec((1, H), lambda b: (0, 0)),
            pl.BlockSpec((_NL, H, H), lambda b: (0, 0, 0)),
            pl.BlockSpec((A, 128), lambda b: (0, 0)),
        ],
        out_specs=pl.BlockSpec((1, A, 7 * H), lambda b: (b, 0, 0)),
        out_shape=jax.ShapeDtypeStruct((B, A, 7 * H), jnp.bfloat16),
        interpret=interpret,
    )(codes, params['W_code'], bc, wm2, apad_j)

    tp = min(256, P)
    kern = functools.partial(_main_kernel, tp=tp, na=A)
    out = pl.pallas_call(
        kern,
        grid=(B, P // tp),
        in_specs=[
            pl.BlockSpec((1, tp, 3), lambda b, i: (b, i, 0)),
            pl.BlockSpec((1, A, 7 * H), lambda b, i: (b, 0, 0)),
            pl.BlockSpec((3, A), lambda b, i: (0, 0)),
            pl.BlockSpec((3, H), lambda b, i: (0, 0)),
            pl.BlockSpec((1, H), lambda b, i: (0, 0)),
            pl.BlockSpec((_NL, H, H), lambda b, i: (0, 0, 0)),
            pl.BlockSpec((_NL, H), lambda b, i: (0, 0)),
            pl.BlockSpec((_NL, H), lambda b, i: (0, 0)),
            pl.BlockSpec((_NL, H), lambda b, i: (0, 0)),
            pl.BlockSpec((_NL, H, H), lambda b, i: (0, 0, 0)),
            pl.BlockSpec((_NL, H, H), lambda b, i: (0, 0, 0)),
            pl.BlockSpec((_NL, H), lambda b, i: (0, 0)),
            pl.BlockSpec((H, nch), lambda b, i: (0, 0)),
            pl.BlockSpec((1, nch), lambda b, i: (0, 0)),
        ],
        out_specs=pl.BlockSpec((1, tp, nch), lambda b, i: (b, i, 0)),
        out_shape=jax.ShapeDtypeStruct((B, P, nch), jnp.float32),
        interpret=interpret,
    )(x, T, arows_j, params['W_q'], bq, wm1, params['b_msg'], wd2, wx,
      wu1, wu2, params['b_upd'], params['W_out'], bout)
    return out


def kernel(x, codes, params):
    return _run(x, codes, params, False)


# speedup vs baseline: 18.9757x; 2.2207x over previous
import functools

import numpy as np
import jax
import jax.numpy as jnp
from jax.experimental import pallas as pl
from jax.experimental.pallas import tpu as pltpu

_GRID_SIZE = 48
_RES = 0.25
_SPACING = 2.0
_H = 128
_K = 16
_NL = 3


def _anchors_np():
    half = (_GRID_SIZE - 1) * _RES / 2.0
    n = int(np.floor(2.0 * half / _SPACING)) + 1
    lin = (np.arange(n) - (n - 1) / 2.0) * _SPACING
    g = np.stack(np.meshgrid(lin, lin, lin, indexing='ij'), axis=-1).reshape(-1, 3)
    return g.astype(np.float32)


def _silu(v):
    return v * (1.0 / (1.0 + jnp.exp(-v)))


def _prep_kernel(codes_ref, wc_ref, bc_ref, wm2_ref, ar_ref, t_ref):
    ct = codes_ref[0]
    hat = _silu(jnp.dot(wc_ref[...], ct, preferred_element_type=jnp.float32)
                + bc_ref[...])
    gs = [jnp.dot(wm2_ref[l], hat, preferred_element_type=jnp.float32)
          for l in range(_NL)]
    G = jnp.concatenate(gs, axis=0)
    t_ref[0] = jnp.concatenate(
        [G.astype(jnp.bfloat16), ar_ref[...].astype(jnp.bfloat16)], axis=0)


def _main_kernel(x_ref, t_ref, ac_ref, wq_ref, bq_ref, wm1_ref, bmsg_ref,
                 wd2_ref, wxr_ref, wu1_ref, wu2_ref, bupd_ref, wout_ref,
                 bout_ref, o_ref, *, tp, na):
    xbt = x_ref[0]
    d2 = None
    for c in range(3):
        t = ac_ref[:, c:c + 1] - xbt[c:c + 1, :]
        t = t * t
        d2 = t if d2 is None else d2 + t
    iota = jax.lax.broadcasted_iota(jnp.int32, (na, tp), 0)
    ib = jax.lax.bitcast_convert_type(d2, jnp.int32)
    key = jax.lax.bitcast_convert_type((ib & jnp.int32(-256)) | iota,
                                       jnp.float32)
    ohs = []
    for _ in range(_K):
        amn = jnp.min(key, axis=0, keepdims=True)
        oh = key == amn
        ohs.append(oh.astype(jnp.bfloat16))
        key = jnp.where(oh, jnp.float32(3.0e38), key)
    OT = jnp.concatenate(ohs, axis=1)
    GAT = jnp.dot(t_ref[0], OT, preferred_element_type=jnp.float32)
    GeT = GAT[0:3 * _H, :]
    NPT = GAT[3 * _H:3 * _H + 3, :]

    hqt = _silu(jnp.dot(wq_ref[...], xbt, preferred_element_type=jnp.float32)
                + bq_ref[...])
    xct = xbt
    inv_k = jnp.float32(1.0 / _K)
    for l in range(_NL):
        qpt = jnp.dot(wm1_ref[l], hqt, preferred_element_type=jnp.float32) \
            + bmsg_ref[l]
        qpe = jnp.concatenate([qpt] * _K, axis=1)
        xce = jnp.concatenate([xct] * _K, axis=1)
        rel = xce - NPT
        d2e = jnp.sum(rel * rel, axis=0, keepdims=True)
        m = _silu(qpe + GeT[l * _H:(l + 1) * _H, :] + wd2_ref[l] * d2e)
        cw = jnp.dot(wxr_ref[l], m, preferred_element_type=jnp.float32)
        rcw = rel * cw
        agg = m[:, 0:tp]
        xup = rcw[:, 0:tp]
        for k in range(1, _K):
            agg = agg + m[:, k * tp:(k + 1) * tp]
            xup = xup + rcw[:, k * tp:(k + 1) * tp]
        agg = agg * inv_k
        xct = xct + xup * inv_k
        hqt = _silu(jnp.dot(wu1_ref[l], hqt, preferred_element_type=jnp.float32)
                    + jnp.dot(wu2_ref[l], agg, preferred_element_type=jnp.float32)
                    + bupd_ref[l])
    o_ref[0] = jnp.dot(wout_ref[...], hqt, preferred_element_type=jnp.float32) \
        + bout_ref[...]


def _run(x, codes, params, interpret=False):
    B, P, _ = x.shape
    A = codes.shape[1]
    H = _H
    anc = _anchors_np()
    acol_j = jnp.asarray(anc)
    arows_j = jnp.asarray(anc.T.copy())

    wmsg = params['W_msg']
    wm1t = wmsg[:, :H, :].transpose(0, 2, 1)
    wm2t = wmsg[:, H:2 * H, :].transpose(0, 2, 1)
    wd2c = wmsg[:, 2 * H, :][:, :, None]
    wupd = params['W_upd']
    wu1t = wupd[:, :H, :].transpose(0, 2, 1)
    wu2t = wupd[:, H:, :].transpose(0, 2, 1)
    wxr = params['W_x'].transpose(0, 2, 1)
    bmsgc = params['b_msg'][:, :, None]
    bupdc = params['b_upd'][:, :, None]
    wqt = params['W_q'].T
    bqc = params['b_q'].reshape(H, 1)
    wct = params['W_code'].T
    bcc = params['b_code'].reshape(H, 1)
    woutt = params['W_out'].T
    boutc = params['b_out'].reshape(-1, 1)
    nch = boutc.shape[0]
    codest = codes.transpose(0, 2, 1)

    TT = pl.pallas_call(
        _prep_kernel,
        grid=(B,),
        in_specs=[
            pl.BlockSpec((1, H, A), lambda b: (b, 0, 0)),
            pl.BlockSpec((H, H), lambda b: (0, 0)),
            pl.BlockSpec((H, 1), lambda b: (0, 0)),
            pl.BlockSpec((_NL, H, H), lambda b: (0, 0, 0)),
            pl.BlockSpec((3, A), lambda b: (0, 0)),
        ],
        out_specs=pl.BlockSpec((1, 3 * H + 3, A), lambda b: (b, 0, 0)),
        out_shape=jax.ShapeDtypeStruct((B, 3 * H + 3, A), jnp.bfloat16),
        interpret=interpret,
    )(codest, wct, bcc, wm2t, arows_j)

    tp = min(512, P)
    kern = functools.partial(_main_kernel, tp=tp, na=A)
    xt = x.transpose(0, 2, 1)
    out_t = pl.pallas_call(
        kern,
        grid=(B, P // tp),
        in_specs=[
            pl.BlockSpec((1, 3, tp), lambda b, i: (b, 0, i)),
            pl.BlockSpec((1, 3 * H + 3, A), lambda b, i: (b, 0, 0)),
            pl.BlockSpec((A, 3), lambda b, i: (0, 0)),
            pl.BlockSpec((H, 3), lambda b, i: (0, 0)),
            pl.BlockSpec((H, 1), lambda b, i: (0, 0)),
            pl.BlockSpec((_NL, H, H), lambda b, i: (0, 0, 0)),
            pl.BlockSpec((_NL, H, 1), lambda b, i: (0, 0, 0)),
            pl.BlockSpec((_NL, H, 1), lambda b, i: (0, 0, 0)),
            pl.BlockSpec((_NL, 1, H), lambda b, i: (0, 0, 0)),
            pl.BlockSpec((_NL, H, H), lambda b, i: (0, 0, 0)),
            pl.BlockSpec((_NL, H, H), lambda b, i: (0, 0, 0)),
            pl.BlockSpec((_NL, H, 1), lambda b, i: (0, 0, 0)),
            pl.BlockSpec((nch, H), lambda b, i: (0, 0)),
            pl.BlockSpec((nch, 1), lambda b, i: (0, 0)),
        ],
        out_specs=pl.BlockSpec((1, nch, tp), lambda b, i: (b, 0, i)),
        out_shape=jax.ShapeDtypeStruct((B, nch, P), jnp.float32),
        interpret=interpret,
    )(xt, TT, acol_j, wqt, bqc, wm1t, bmsgc, wd2c, wxr,
      wu1t, wu2t, bupdc, woutt, boutc)
    return out_t.transpose(0, 2, 1)


def kernel(x, codes, params):
    return _run(x, codes, params, False)


# speedup vs baseline: 20.5644x; 1.0837x over previous
import functools

import numpy as np
import jax
import jax.numpy as jnp
from jax.experimental import pallas as pl
from jax.experimental.pallas import tpu as pltpu

_GRID_SIZE = 48
_RES = 0.25
_SPACING = 2.0
_H = 128
_K = 16
_NL = 3


def _anchors_np():
    half = (_GRID_SIZE - 1) * _RES / 2.0
    n = int(np.floor(2.0 * half / _SPACING)) + 1
    lin = (np.arange(n) - (n - 1) / 2.0) * _SPACING
    g = np.stack(np.meshgrid(lin, lin, lin, indexing='ij'), axis=-1).reshape(-1, 3)
    return g.astype(np.float32)


def _silu(v):
    return v * (0.5 * jnp.tanh(0.5 * v) + 0.5)


def _prep_kernel(codes_ref, wc_ref, bc_ref, wm2_ref, wd2_ref, t_ref):
    ct = codes_ref[0]
    hat = _silu(jnp.dot(wc_ref[...], ct, preferred_element_type=jnp.float32)
                + bc_ref[...])
    for l in range(_NL):
        g = jnp.dot(wm2_ref[l], hat, preferred_element_type=jnp.float32)
        wd2b = wd2_ref[l].astype(jnp.bfloat16)
        t_ref[0, l] = jnp.concatenate(
            [g.astype(jnp.bfloat16), wd2b, wd2b], axis=1)


def _main_kernel(x_ref, t_ref, ac_ref, ar_ref, wq_ref, bq_ref, wm1_ref,
                 bmsg_ref, wxr_ref, wu1_ref, wu2_ref, bupd_ref, wout_ref,
                 bout_ref, o_ref, s_ref, *, tp, na):
    xbt = x_ref[0]
    d2 = None
    for c in range(3):
        t = ac_ref[:, c:c + 1] - xbt[c:c + 1, :]
        t = t * t
        d2 = t if d2 is None else d2 + t
    iota = jax.lax.broadcasted_iota(jnp.int32, (na, tp), 0)
    ib = jax.lax.bitcast_convert_type(d2, jnp.int32)
    key = jax.lax.bitcast_convert_type((ib & jnp.int32(-256)) | iota,
                                       jnp.float32)
    for k in range(_K):
        amn = jnp.min(key, axis=0, keepdims=True)
        oh = key == amn
        s_ref[0:na, k * tp:(k + 1) * tp] = oh.astype(jnp.bfloat16)
        key = jnp.where(oh, jnp.float32(3.0e38), key)
    NPT = jnp.dot(ar_ref[...], s_ref[0:na, :],
                  preferred_element_type=jnp.float32)

    hqt = _silu(jnp.dot(wq_ref[...], xbt, preferred_element_type=jnp.float32)
                + bq_ref[...])
    xct = xbt
    inv_k = jnp.float32(1.0 / _K)
    for l in range(_NL):
        qpt = jnp.dot(wm1_ref[l], hqt, preferred_element_type=jnp.float32) \
            + bmsg_ref[l]
        qpe = jnp.concatenate([qpt] * _K, axis=1)
        xce = jnp.concatenate([xct] * _K, axis=1)
        rel = xce - NPT
        d2e = jnp.sum(rel * rel, axis=0, keepdims=True)
        d2h = d2e.astype(jnp.bfloat16)
        d2l = (d2e - d2h.astype(jnp.float32)).astype(jnp.bfloat16)
        s_ref[na:na + 2, :] = jnp.concatenate([d2h, d2l], axis=0)
        ge = jnp.dot(t_ref[0, l], s_ref[0:na + 2, :],
                     preferred_element_type=jnp.float32)
        m = _silu(qpe + ge)
        cw = jnp.dot(wxr_ref[l], m, preferred_element_type=jnp.float32)
        rcw = rel * cw
        agg = m[:, 0:tp]
        xup = rcw[:, 0:tp]
        for k in range(1, _K):
            agg = agg + m[:, k * tp:(k + 1) * tp]
            xup = xup + rcw[:, k * tp:(k + 1) * tp]
        agg = agg * inv_k
        xct = xct + xup * inv_k
        hqt = _silu(jnp.dot(wu1_ref[l], hqt, preferred_element_type=jnp.float32)
                    + jnp.dot(wu2_ref[l], agg, preferred_element_type=jnp.float32)
                    + bupd_ref[l])
    o_ref[0] = jnp.dot(wout_ref[...], hqt, preferred_element_type=jnp.float32) \
        + bout_ref[...]


def _run(x, codes, params, interpret=False):
    B, P, _ = x.shape
    A = codes.shape[1]
    H = _H
    anc = _anchors_np()
    acol_j = jnp.asarray(anc)
    arows_j = jnp.asarray(anc.T.copy())

    wmsg = params['W_msg']
    wm1t = wmsg[:, :H, :].transpose(0, 2, 1)
    wm2t = wmsg[:, H:2 * H, :].transpose(0, 2, 1)
    wd2c = wmsg[:, 2 * H, :][:, :, None]
    wupd = params['W_upd']
    wu1t = wupd[:, :H, :].transpose(0, 2, 1)
    wu2t = wupd[:, H:, :].transpose(0, 2, 1)
    wxr = params['W_x'].transpose(0, 2, 1)
    bmsgc = params['b_msg'][:, :, None]
    bupdc = params['b_upd'][:, :, None]
    wqt = params['W_q'].T
    bqc = params['b_q'].reshape(H, 1)
    wct = params['W_code'].T
    bcc = params['b_code'].reshape(H, 1)
    woutt = params['W_out'].T
    boutc = params['b_out'].reshape(-1, 1)
    nch = boutc.shape[0]
    codest = codes.transpose(0, 2, 1)

    TL = pl.pallas_call(
        _prep_kernel,
        grid=(B,),
        in_specs=[
            pl.BlockSpec((1, H, A), lambda b: (b, 0, 0)),
            pl.BlockSpec((H, H), lambda b: (0, 0)),
            pl.BlockSpec((H, 1), lambda b: (0, 0)),
            pl.BlockSpec((_NL, H, H), lambda b: (0, 0, 0)),
            pl.BlockSpec((_NL, H, 1), lambda b: (0, 0, 0)),
        ],
        out_specs=pl.BlockSpec((1, _NL, H, A + 2), lambda b: (b, 0, 0, 0)),
        out_shape=jax.ShapeDtypeStruct((B, _NL, H, A + 2), jnp.bfloat16),
        interpret=interpret,
    )(codest, wct, bcc, wm2t, wd2c)

    tp = min(1024, P)
    kern = functools.partial(_main_kernel, tp=tp, na=A)
    xt = x.transpose(0, 2, 1)
    arbf_j = arows_j.astype(jnp.bfloat16)
    out_t = pl.pallas_call(
        kern,
        grid=(B, P // tp),
        in_specs=[
            pl.BlockSpec((1, 3, tp), lambda b, i: (b, 0, i)),
            pl.BlockSpec((1, _NL, H, A + 2), lambda b, i: (b, 0, 0, 0)),
            pl.BlockSpec((A, 3), lambda b, i: (0, 0)),
            pl.BlockSpec((3, A), lambda b, i: (0, 0)),
            pl.BlockSpec((H, 3), lambda b, i: (0, 0)),
            pl.BlockSpec((H, 1), lambda b, i: (0, 0)),
            pl.BlockSpec((_NL, H, H), lambda b, i: (0, 0, 0)),
            pl.BlockSpec((_NL, H, 1), lambda b, i: (0, 0, 0)),
            pl.BlockSpec((_NL, 1, H), lambda b, i: (0, 0, 0)),
            pl.BlockSpec((_NL, H, H), lambda b, i: (0, 0, 0)),
            pl.BlockSpec((_NL, H, H), lambda b, i: (0, 0, 0)),
            pl.BlockSpec((_NL, H, 1), lambda b, i: (0, 0, 0)),
            pl.BlockSpec((nch, H), lambda b, i: (0, 0)),
            pl.BlockSpec((nch, 1), lambda b, i: (0, 0)),
        ],
        out_specs=pl.BlockSpec((1, nch, tp), lambda b, i: (b, 0, i)),
        out_shape=jax.ShapeDtypeStruct((B, nch, P), jnp.float32),
        scratch_shapes=[pltpu.VMEM((A + 8, _K * tp), jnp.bfloat16)],
        interpret=interpret,
    )(xt, TL, acol_j, arbf_j, wqt, bqc, wm1t, bmsgc, wxr,
      wu1t, wu2t, bupdc, woutt, boutc)
    return out_t.transpose(0, 2, 1)


def kernel(x, codes, params):
    return _run(x, codes, params, False)
